# Initial kernel scaffold; baseline (speedup 1.0000x reference)
import jax
import jax.numpy as jnp
from jax.experimental import pallas as pl


def kernel(x, edge_index, Wl1, bl1, Wr1, Wl2, bl2, Wr2, Wlin, blin):
    raise NotImplementedError("write your pallas kernel here")


# speedup vs baseline: 1.6009x; 1.6009x over previous
import functools

import jax
import jax.numpy as jnp
from jax import lax
from jax.experimental import pallas as pl
from jax.experimental.pallas import tpu as pltpu
from jax.experimental.pallas import tpu_sc as plsc

N = 10000
E = 320000
D = 128
NC = 2
NS = 16
B = 128
EPAD = 327680
NBATCH = EPAD // B
TPB = NBATCH // NS
H = 5120
NR = 2
NOUT = H * NR
HPT = H // NS
ACC = 5248
APT = ACC // NS
DLOC = H
DUMMY = NOUT

_mesh1 = plsc.VectorSubcoreMesh(core_axis_name="c", subcore_axis_name="s",
                                num_cores=1)
_mesh2 = plsc.VectorSubcoreMesh(core_axis_name="c", subcore_axis_name="s")


def _zero_2d(ref, rows, width):
    z = jnp.zeros((16,), jnp.float32)

    def row(i, carry):
        for j in range(width // 16):
            ref[i, pl.ds(j * 16, 16)] = z
        return carry

    lax.fori_loop(0, rows, row, 0)


def _zero_1d(ref, n):
    z = jnp.zeros((16,), jnp.float32)

    def body(i, carry):
        ref[pl.ds(i * 16, 16)] = z
        return carry

    lax.fori_loop(0, n // 16, body, 0)


def _agg_round(r, x_hbm, src_hbm, dst_hbm, out_hbm, deg_hbm, sidx, didx,
               rows, wbuf, accum, sem, deg_part):
    s = lax.axis_index("s")
    lo = r * H

    pltpu.sync_copy(wbuf.at[pl.ds(0, APT)], accum.at[pl.ds(s * APT, APT)])
    if deg_part is not None:
        ones, dbuf, dacc = deg_part

        @pl.when(s == 0)
        def _():
            pltpu.sync_copy(dbuf, dacc)

    plsc.subcore_barrier()

    base = s * TPB

    def batch(b, carry):
        row = base + b
        pltpu.sync_copy(src_hbm.at[row], sidx)
        pltpu.sync_copy(dst_hbm.at[row], didx)
        for j in range(B // 16):
            sl = pl.ds(j * 16, 16)
            u = didx[sl]
            m = (u >= lo) & (u < lo + H)
            didx[sl] = jnp.where(m, u - lo, DLOC)
        pltpu.async_copy(x_hbm.at[sidx], rows, sem).wait()
        pltpu.sync_copy(rows, accum.at[didx], add=True)
        if deg_part is not None:
            ones, _, dacc = deg_part
            pltpu.sync_copy(ones, dacc.at[didx], add=True)
        return carry

    lax.fori_loop(0, TPB, batch, 0)

    plsc.subcore_barrier()

    pltpu.sync_copy(accum.at[pl.ds(s * HPT, HPT)], wbuf.at[pl.ds(0, HPT)])
    pltpu.sync_copy(wbuf.at[pl.ds(0, HPT)],
                    out_hbm.at[pl.ds(lo + s * HPT, HPT)])
    if deg_part is not None:
        ones, dbuf, dacc = deg_part
        pltpu.sync_copy(dacc.at[pl.ds(s * HPT, HPT)], dbuf.at[pl.ds(0, HPT)])
        pltpu.sync_copy(dbuf.at[pl.ds(0, HPT)],
                        deg_hbm.at[pl.ds(lo + s * HPT, HPT)])
    plsc.subcore_barrier()
    _zero_2d(wbuf, APT, D)
    if deg_part is not None:
        ones, dbuf, dacc = deg_part
        _zero_1d(dbuf, ACC)


@functools.partial(
    pl.kernel,
    out_type=[
        jax.ShapeDtypeStruct((NOUT, D), jnp.float32),
        jax.ShapeDtypeStruct((NOUT,), jnp.float32),
    ],
    mesh=_mesh1,
    scratch_types=[
        pltpu.VMEM((B,), jnp.int32),
        pltpu.VMEM((B,), jnp.int32),
        pltpu.VMEM((B, D), jnp.float32),
        pltpu.VMEM((APT, D), jnp.float32),
        pltpu.VMEM_SHARED((ACC, D), jnp.float32),
        pltpu.SemaphoreType.DMA,
        pltpu.VMEM((B,), jnp.float32),
        pltpu.VMEM((ACC,), jnp.float32),
        pltpu.VMEM_SHARED((ACC,), jnp.float32),
    ],
)
def _agg_deg(x_hbm, src_hbm, dst_hbm, out_hbm, deg_hbm, sidx, didx, rows,
             wbuf, accum, sem, ones, dbuf, dacc):
    one = jnp.full((16,), 1.0, jnp.float32)
    for j in range(B // 16):
        ones[pl.ds(j * 16, 16)] = one
    _zero_2d(wbuf, APT, D)
    _zero_1d(dbuf, ACC)
    for r in range(NR):
        _agg_round(r, x_hbm, src_hbm, dst_hbm, out_hbm, deg_hbm, sidx, didx,
                   rows, wbuf, accum, sem, (ones, dbuf, dacc))


@functools.partial(
    pl.kernel,
    out_type=jax.ShapeDtypeStruct((NOUT, D), jnp.float32),
    mesh=_mesh1,
    scratch_types=[
        pltpu.VMEM((B,), jnp.int32),
        pltpu.VMEM((B,), jnp.int32),
        pltpu.VMEM((B, D), jnp.float32),
        pltpu.VMEM((APT, D), jnp.float32),
        pltpu.VMEM_SHARED((ACC, D), jnp.float32),
        pltpu.SemaphoreType.DMA,
    ],
)
def _agg(x_hbm, src_hbm, dst_hbm, out_hbm, sidx, didx, rows, wbuf, accum,
         sem):
    _zero_2d(wbuf, APT, D)
    for r in range(NR):
        _agg_round(r, x_hbm, src_hbm, dst_hbm, out_hbm, None, sidx, didx,
                   rows, wbuf, accum, sem, None)


@functools.partial(
    pl.kernel,
    out_type=jax.ShapeDtypeStruct((EPAD,), jnp.float32),
    mesh=_mesh2,
    scratch_types=[
        pltpu.VMEM((B,), jnp.int32),
        pltpu.VMEM((B,), jnp.int32),
        pltpu.VMEM((B,), jnp.float32),
        pltpu.VMEM((B,), jnp.float32),
        pltpu.VMEM((B,), jnp.float32),
        pltpu.SemaphoreType.DMA,
    ],
)
def _decode(s_hbm, t_hbm, src_hbm, dst_hbm, out_hbm, sidx, didx, sg, tg, ob,
            sem):
    c = lax.axis_index("c")
    s = lax.axis_index("s")
    base = (c * NS + s) * (NBATCH // (NC * NS))

    def batch(b, carry):
        row = base + b
        pltpu.sync_copy(src_hbm.at[row], sidx)
        pltpu.sync_copy(dst_hbm.at[row], didx)
        pltpu.async_copy(s_hbm.at[sidx], sg, sem).wait()
        pltpu.async_copy(t_hbm.at[didx], tg, sem).wait()

        def lane(j, carry2):
            v = sg[pl.ds(j * 16, 16)] + tg[pl.ds(j * 16, 16)]
            ob[pl.ds(j * 16, 16)] = 1.0 / (1.0 + jnp.exp(-v))
            return carry2

        lax.fori_loop(0, B // 16, lane, 0)
        pltpu.sync_copy(ob, out_hbm.at[pl.ds(row * B, B)])
        return carry

    lax.fori_loop(0, NBATCH // (NC * NS), batch, 0)


RB = 2000
GRID = N // RB


def _tc1_body(p_ref, dg_ref, x_ref, wl_ref, wr_ref, bl_ref, o_ref):
    inv = 1.0 / jnp.maximum(dg_ref[...], 1.0)
    agg = p_ref[...] * inv
    h = lax.dot_general(agg, wl_ref[...], (((1,), (1,)), ((), ())),
                        preferred_element_type=jnp.float32)
    h = h + bl_ref[...]
    h = h + lax.dot_general(x_ref[...], wr_ref[...], (((1,), (1,)), ((), ())),
                            preferred_element_type=jnp.float32)
    o_ref[...] = jnp.maximum(h, 0.0)


def _tc1(p, dg, x, Wl1, bl1, Wr1):
    return pl.pallas_call(
        _tc1_body,
        grid=(GRID,),
        in_specs=[
            pl.BlockSpec((RB, D), lambda i: (i, 0)),
            pl.BlockSpec((RB, 1), lambda i: (i, 0)),
            pl.BlockSpec((RB, D), lambda i: (i, 0)),
            pl.BlockSpec((D, D), lambda i: (0, 0)),
            pl.BlockSpec((D, D), lambda i: (0, 0)),
            pl.BlockSpec((1, D), lambda i: (0, 0)),
        ],
        out_specs=pl.BlockSpec((RB, D), lambda i: (i, 0)),
        out_shape=jax.ShapeDtypeStruct((N, D), jnp.float32),
    )(p, dg, x, Wl1, Wr1, bl1)


def _tc2_body(p_ref, dg_ref, h_ref, wl_ref, wr_ref, bl_ref, wd_ref, bv_ref,
              o_ref):
    inv = 1.0 / jnp.maximum(dg_ref[...], 1.0)
    agg = p_ref[...] * inv
    z = lax.dot_general(agg, wl_ref[...], (((1,), (1,)), ((), ())),
                        preferred_element_type=jnp.float32)
    z = z + bl_ref[...]
    z = z + lax.dot_general(h_ref[...], wr_ref[...], (((1,), (1,)), ((), ())),
                            preferred_element_type=jnp.float32)
    o_ref[...] = lax.dot_general(z, wd_ref[...], (((1,), (0,)), ((), ())),
                                 preferred_element_type=jnp.float32) + bv_ref[...]


def _tc2(p, dg, h, Wl2, bl2, Wr2, wd, bv):
    return pl.pallas_call(
        _tc2_body,
        grid=(GRID,),
        in_specs=[
            pl.BlockSpec((RB, D), lambda i: (i, 0)),
            pl.BlockSpec((RB, 1), lambda i: (i, 0)),
            pl.BlockSpec((RB, D), lambda i: (i, 0)),
            pl.BlockSpec((D, D), lambda i: (0, 0)),
            pl.BlockSpec((D, D), lambda i: (0, 0)),
            pl.BlockSpec((1, D), lambda i: (0, 0)),
            pl.BlockSpec((D, 2), lambda i: (0, 0)),
            pl.BlockSpec((1, 2), lambda i: (0, 0)),
        ],
        out_specs=pl.BlockSpec((RB, 2), lambda i: (i, 0)),
        out_shape=jax.ShapeDtypeStruct((N, 2), jnp.float32),
    )(p, dg, h, Wl2, Wr2, bl2, wd, bv)


def kernel(x, edge_index, Wl1, bl1, Wr1, Wl2, bl2, Wr2, Wlin, blin):
    src = edge_index[0].astype(jnp.int32)
    dst = edge_index[1].astype(jnp.int32)
    pad = EPAD - E
    src2d = jnp.concatenate([src, jnp.zeros((pad,), jnp.int32)]).reshape(
        NBATCH, B)
    dst2d = jnp.concatenate([dst, jnp.full((pad,), DUMMY, jnp.int32)]).reshape(
        NBATCH, B)

    p1, deg = _agg_deg(x, src2d, dst2d)
    dg = deg[:N].reshape(N, 1)
    h = _tc1(p1, dg, x, Wl1, bl1.reshape(1, D), Wr1)

    p2 = _agg(h, src2d, dst2d)
    wd = jnp.stack([Wlin[0, :D], Wlin[0, D:]], axis=1)
    bv = jnp.stack([blin[0], jnp.zeros((), jnp.float32)]).reshape(1, 2)
    st = _tc2(p2, dg, h, Wl2, bl2.reshape(1, D), Wr2, wd, bv)

    s_tab = jnp.pad(st[:, 0], (0, NOUT - N))
    t_tab = jnp.pad(st[:, 1], (0, NOUT - N))
    logits = _decode(s_tab, t_tab, src2d, dst2d)
    return logits[:E]


# speedup vs baseline: 2.1467x; 1.3410x over previous
import functools

import jax
import jax.numpy as jnp
from jax import lax
from jax.experimental import pallas as pl
from jax.experimental.pallas import tpu as pltpu
from jax.experimental.pallas import tpu_sc as plsc

N = 10000
E = 320000
D = 128
NC = 2
NS = 16
B = 128
EPAD = 327680
NBATCH = EPAD // B
TPB = NBATCH // NS
H = 5120
NR = 2
NOUT = H * NR
HPT = H // NS
ACC = 5248
APT = ACC // NS
DLOC = H
DUMMY = NOUT
NBUF = 2

_mesh1 = plsc.VectorSubcoreMesh(core_axis_name="c", subcore_axis_name="s",
                                num_cores=1)
_mesh2 = plsc.VectorSubcoreMesh(core_axis_name="c", subcore_axis_name="s")


def _zero_2d(ref, rows, width):
    z = jnp.zeros((16,), jnp.float32)

    def row(i, carry):
        for j in range(width // 16):
            ref[i, pl.ds(j * 16, 16)] = z
        return carry

    lax.fori_loop(0, rows, row, 0)


def _zero_1d(ref, n):
    z = jnp.zeros((16,), jnp.float32)

    def body(i, carry):
        ref[pl.ds(i * 16, 16)] = z
        return carry

    lax.fori_loop(0, n // 16, body, 0)


def _agg_round(r, x_hbm, src_hbm, dst_hbm, out_hbm, deg_hbm, sidx, didxr,
               didx, rows, esem, gsem, ssem, wbuf, accum, deg_part):
    s = lax.axis_index("s")
    lo = r * H

    pltpu.sync_copy(wbuf.at[pl.ds(0, APT)], accum.at[pl.ds(s * APT, APT)])
    if deg_part is not None:
        ones, dbuf, dacc = deg_part

        @pl.when(s == 0)
        def _():
            pltpu.sync_copy(dbuf, dacc)

    plsc.subcore_barrier()

    base = s * TPB

    def _drain(k):
        pltpu.make_async_copy(rows[k], accum.at[didx[k]], ssem[k]).wait()
        if deg_part is not None:
            ones, _, dacc = deg_part
            pltpu.make_async_copy(ones, dacc.at[didx[k]], ssem[k]).wait()

    def _load_idx(b, k):
        pltpu.async_copy(src_hbm.at[b], sidx[k], esem[k])
        pltpu.async_copy(dst_hbm.at[b], didxr[k], esem[k])

    def _wait_idx(b, k):
        pltpu.make_async_copy(src_hbm.at[b], sidx[k], esem[k]).wait()
        pltpu.make_async_copy(dst_hbm.at[b], didxr[k], esem[k]).wait()

    for k in range(NBUF):
        _load_idx(base + k, k)

    def batch(i, carry):
        for k in range(NBUF):
            b = i * NBUF + k

            @pl.when(i > 0)
            def _():
                _drain(k)

            _wait_idx(base + b, k)
            for j in range(B // 16):
                sl = pl.ds(j * 16, 16)
                u = didxr[k][sl]
                m = (u >= lo) & (u < lo + H)
                didx[k][sl] = jnp.where(m, u - lo, DLOC)
            pltpu.async_copy(x_hbm.at[sidx[k]], rows[k], gsem[k])

        for k in range(NBUF):
            b = i * NBUF + k
            pltpu.make_async_copy(x_hbm.at[sidx[k]], rows[k],
                                  gsem[k]).wait()

            @pl.when(b + NBUF < TPB)
            def _():
                _load_idx(base + b + NBUF, k)

            desc = pltpu.make_async_copy(rows[k], accum.at[didx[k]], ssem[k])
            desc.start(add=True)
            if deg_part is not None:
                ones, _, dacc = deg_part
                d2 = pltpu.make_async_copy(ones, dacc.at[didx[k]], ssem[k])
                d2.start(add=True)
        return carry

    lax.fori_loop(0, TPB // NBUF, batch, 0)
    for k in range(NBUF):
        _drain(k)

    plsc.subcore_barrier()

    pltpu.sync_copy(accum.at[pl.ds(s * HPT, HPT)], wbuf.at[pl.ds(0, HPT)])
    pltpu.sync_copy(wbuf.at[pl.ds(0, HPT)],
                    out_hbm.at[pl.ds(lo + s * HPT, HPT)])
    if deg_part is not None:
        ones, dbuf, dacc = deg_part
        pltpu.sync_copy(dacc.at[pl.ds(s * HPT, HPT)], dbuf.at[pl.ds(0, HPT)])
        pltpu.sync_copy(dbuf.at[pl.ds(0, HPT)],
                        deg_hbm.at[pl.ds(lo + s * HPT, HPT)])
    plsc.subcore_barrier()
    _zero_2d(wbuf, APT, D)
    if deg_part is not None:
        ones, dbuf, dacc = deg_part
        _zero_1d(dbuf, ACC)


_AGG_SCRATCH = (
    [pltpu.VMEM((B,), jnp.int32) for _ in range(3 * NBUF)]
    + [pltpu.VMEM((B, D), jnp.float32) for _ in range(NBUF)]
    + [pltpu.SemaphoreType.DMA for _ in range(3 * NBUF)]
    + [
        pltpu.VMEM((APT, D), jnp.float32),
        pltpu.VMEM_SHARED((ACC, D), jnp.float32),
    ]
)


@functools.partial(
    pl.kernel,
    out_type=[
        jax.ShapeDtypeStruct((NOUT, D), jnp.float32),
        jax.ShapeDtypeStruct((NOUT,), jnp.float32),
    ],
    mesh=_mesh1,
    scratch_types=_AGG_SCRATCH + [
        pltpu.VMEM((B,), jnp.float32),
        pltpu.VMEM((ACC,), jnp.float32),
        pltpu.VMEM_SHARED((ACC,), jnp.float32),
    ],
)
def _agg_deg(x_hbm, src_hbm, dst_hbm, out_hbm, deg_hbm, *refs):
    sidx = refs[0:NBUF]
    didxr = refs[NBUF:2 * NBUF]
    didx = refs[2 * NBUF:3 * NBUF]
    rows = refs[3 * NBUF:4 * NBUF]
    esem = refs[4 * NBUF:5 * NBUF]
    gsem = refs[5 * NBUF:6 * NBUF]
    ssem = refs[6 * NBUF:7 * NBUF]
    wbuf, accum, ones, dbuf, dacc = refs[7 * NBUF:]

    one = jnp.full((16,), 1.0, jnp.float32)
    for j in range(B // 16):
        ones[pl.ds(j * 16, 16)] = one
    _zero_2d(wbuf, APT, D)
    _zero_1d(dbuf, ACC)
    for r in range(NR):
        _agg_round(r, x_hbm, src_hbm, dst_hbm, out_hbm, deg_hbm, sidx, didxr,
                   didx, rows, esem, gsem, ssem, wbuf, accum,
                   (ones, dbuf, dacc))


@functools.partial(
    pl.kernel,
    out_type=jax.ShapeDtypeStruct((NOUT, D), jnp.float32),
    mesh=_mesh1,
    scratch_types=_AGG_SCRATCH,
)
def _agg(x_hbm, src_hbm, dst_hbm, out_hbm, *refs):
    sidx = refs[0:NBUF]
    didxr = refs[NBUF:2 * NBUF]
    didx = refs[2 * NBUF:3 * NBUF]
    rows = refs[3 * NBUF:4 * NBUF]
    esem = refs[4 * NBUF:5 * NBUF]
    gsem = refs[5 * NBUF:6 * NBUF]
    ssem = refs[6 * NBUF:7 * NBUF]
    wbuf, accum = refs[7 * NBUF:]

    _zero_2d(wbuf, APT, D)
    for r in range(NR):
        _agg_round(r, x_hbm, src_hbm, dst_hbm, out_hbm, None, sidx, didxr,
                   didx, rows, esem, gsem, ssem, wbuf, accum, None)


@functools.partial(
    pl.kernel,
    out_type=jax.ShapeDtypeStruct((EPAD,), jnp.float32),
    mesh=_mesh2,
    scratch_types=(
        [pltpu.VMEM((B,), jnp.int32) for _ in range(2 * NBUF)]
        + [pltpu.VMEM((B,), jnp.float32) for _ in range(3 * NBUF)]
        + [pltpu.SemaphoreType.DMA for _ in range(3 * NBUF)]
    ),
)
def _decode(s_hbm, t_hbm, src_hbm, dst_hbm, out_hbm, *refs):
    sidx = refs[0:NBUF]
    didx = refs[NBUF:2 * NBUF]
    sg = refs[2 * NBUF:3 * NBUF]
    tg = refs[3 * NBUF:4 * NBUF]
    ob = refs[4 * NBUF:5 * NBUF]
    esem = refs[5 * NBUF:6 * NBUF]
    gsem = refs[6 * NBUF:7 * NBUF]
    osem = refs[7 * NBUF:8 * NBUF]

    c = lax.axis_index("c")
    s = lax.axis_index("s")
    tpb = NBATCH // (NC * NS)
    base = (c * NS + s) * tpb

    def _load_idx(b, k):
        pltpu.async_copy(src_hbm.at[b], sidx[k], esem[k])
        pltpu.async_copy(dst_hbm.at[b], didx[k], esem[k])

    def _wait_idx(b, k):
        pltpu.make_async_copy(src_hbm.at[b], sidx[k], esem[k]).wait()
        pltpu.make_async_copy(dst_hbm.at[b], didx[k], esem[k]).wait()

    for k in range(NBUF):
        _load_idx(base + k, k)

    def batch(i, carry):
        for k in range(NBUF):
            b = i * NBUF + k
            _wait_idx(base + b, k)
            pltpu.async_copy(s_hbm.at[sidx[k]], sg[k], gsem[k])
            pltpu.async_copy(t_hbm.at[didx[k]], tg[k], gsem[k])
        for k in range(NBUF):
            b = i * NBUF + k
            pltpu.make_async_copy(s_hbm.at[sidx[k]], sg[k], gsem[k]).wait()
            pltpu.make_async_copy(t_hbm.at[didx[k]], tg[k], gsem[k]).wait()

            @pl.when(i > 0)
            def _():
                pltpu.make_async_copy(
                    ob[k], out_hbm.at[pl.ds(0, B)], osem[k]).wait()

            for j in range(B // 16):
                sl = pl.ds(j * 16, 16)
                v = sg[k][sl] + tg[k][sl]
                ob[k][sl] = 1.0 / (1.0 + jnp.exp(-v))

            @pl.when(b + NBUF < tpb)
            def _():
                _load_idx(base + b + NBUF, k)
            pltpu.async_copy(ob[k], out_hbm.at[pl.ds((base + b) * B, B)],
                             osem[k])
        return carry

    lax.fori_loop(0, tpb // NBUF, batch, 0)
    for k in range(NBUF):
        pltpu.make_async_copy(ob[k], out_hbm.at[pl.ds(0, B)], osem[k]).wait()


RB = 2000
GRID = N // RB


def _tc1_body(p_ref, dg_ref, x_ref, wl_ref, wr_ref, bl_ref, o_ref):
    inv = 1.0 / jnp.maximum(dg_ref[...], 1.0)
    agg = p_ref[...] * inv
    h = lax.dot_general(agg, wl_ref[...], (((1,), (1,)), ((), ())),
                        preferred_element_type=jnp.float32)
    h = h + bl_ref[...]
    h = h + lax.dot_general(x_ref[...], wr_ref[...], (((1,), (1,)), ((), ())),
                            preferred_element_type=jnp.float32)
    o_ref[...] = jnp.maximum(h, 0.0)


def _tc1(p, dg, x, Wl1, bl1, Wr1):
    return pl.pallas_call(
        _tc1_body,
        grid=(GRID,),
        in_specs=[
            pl.BlockSpec((RB, D), lambda i: (i, 0)),
            pl.BlockSpec((RB, 1), lambda i: (i, 0)),
            pl.BlockSpec((RB, D), lambda i: (i, 0)),
            pl.BlockSpec((D, D), lambda i: (0, 0)),
            pl.BlockSpec((D, D), lambda i: (0, 0)),
            pl.BlockSpec((1, D), lambda i: (0, 0)),
        ],
        out_specs=pl.BlockSpec((RB, D), lambda i: (i, 0)),
        out_shape=jax.ShapeDtypeStruct((N, D), jnp.float32),
    )(p, dg, x, Wl1, Wr1, bl1)


def _tc2_body(p_ref, dg_ref, h_ref, wl_ref, wr_ref, bl_ref, wd_ref, bv_ref,
              o_ref):
    inv = 1.0 / jnp.maximum(dg_ref[...], 1.0)
    agg = p_ref[...] * inv
    z = lax.dot_general(agg, wl_ref[...], (((1,), (1,)), ((), ())),
                        preferred_element_type=jnp.float32)
    z = z + bl_ref[...]
    z = z + lax.dot_general(h_ref[...], wr_ref[...], (((1,), (1,)), ((), ())),
                            preferred_element_type=jnp.float32)
    o_ref[...] = lax.dot_general(z, wd_ref[...], (((1,), (0,)), ((), ())),
                                 preferred_element_type=jnp.float32) + bv_ref[...]


def _tc2(p, dg, h, Wl2, bl2, Wr2, wd, bv):
    return pl.pallas_call(
        _tc2_body,
        grid=(GRID,),
        in_specs=[
            pl.BlockSpec((RB, D), lambda i: (i, 0)),
            pl.BlockSpec((RB, 1), lambda i: (i, 0)),
            pl.BlockSpec((RB, D), lambda i: (i, 0)),
            pl.BlockSpec((D, D), lambda i: (0, 0)),
            pl.BlockSpec((D, D), lambda i: (0, 0)),
            pl.BlockSpec((1, D), lambda i: (0, 0)),
            pl.BlockSpec((D, 2), lambda i: (0, 0)),
            pl.BlockSpec((1, 2), lambda i: (0, 0)),
        ],
        out_specs=pl.BlockSpec((RB, 2), lambda i: (i, 0)),
        out_shape=jax.ShapeDtypeStruct((N, 2), jnp.float32),
    )(p, dg, h, Wl2, Wr2, bl2, wd, bv)


def kernel(x, edge_index, Wl1, bl1, Wr1, Wl2, bl2, Wr2, Wlin, blin):
    src = edge_index[0].astype(jnp.int32)
    dst = edge_index[1].astype(jnp.int32)
    pad = EPAD - E
    src2d = jnp.concatenate([src, jnp.zeros((pad,), jnp.int32)]).reshape(
        NBATCH, B)
    dst2d = jnp.concatenate([dst, jnp.full((pad,), DUMMY, jnp.int32)]).reshape(
        NBATCH, B)

    p1, deg = _agg_deg(x, src2d, dst2d)
    dg = deg[:N].reshape(N, 1)
    h = _tc1(p1, dg, x, Wl1, bl1.reshape(1, D), Wr1)

    p2 = _agg(h, src2d, dst2d)
    wd = jnp.stack([Wlin[0, :D], Wlin[0, D:]], axis=1)
    bv = jnp.stack([blin[0], jnp.zeros((), jnp.float32)]).reshape(1, 2)
    st = _tc2(p2, dg, h, Wl2, bl2.reshape(1, D), Wr2, wd, bv)

    s_tab = jnp.pad(st[:, 0], (0, NOUT - N))
    t_tab = jnp.pad(st[:, 1], (0, NOUT - N))
    logits = _decode(s_tab, t_tab, src2d, dst2d)
    return logits[:E]


# speedup vs baseline: 2.1475x; 1.0004x over previous
import functools

import jax
import jax.numpy as jnp
from jax import lax
from jax.experimental import pallas as pl
from jax.experimental.pallas import tpu as pltpu
from jax.experimental.pallas import tpu_sc as plsc

N = 10000
E = 320000
D = 128
NC = 2
NS = 16
B = 128
EPAD = 327680
NBATCH = EPAD // B
TPB = NBATCH // NS
H = 5120
NR = 2
NOUT = H * NR
HPT = H // NS
ACC = 5248
APT = ACC // NS
DLOC = H
DUMMY = NOUT
NBUF = 2

_mesh1 = plsc.VectorSubcoreMesh(core_axis_name="c", subcore_axis_name="s",
                                num_cores=1)
_mesh2 = plsc.VectorSubcoreMesh(core_axis_name="c", subcore_axis_name="s")


def _zero_2d(ref, rows, width):
    z = jnp.zeros((16,), jnp.float32)

    def row(i, carry):
        for j in range(width // 16):
            ref[i, pl.ds(j * 16, 16)] = z
        return carry

    lax.fori_loop(0, rows, row, 0)


def _zero_1d(ref, n):
    z = jnp.zeros((16,), jnp.float32)

    def body(i, carry):
        ref[pl.ds(i * 16, 16)] = z
        return carry

    lax.fori_loop(0, n // 16, body, 0)


def _agg_round(r, x_hbm, src_hbm, dst_hbm, out_hbm, deg_hbm, sidx, didxr,
               didx, rows, esem, gsem, ssem, wbuf, accum, deg_part):
    s = lax.axis_index("s")
    lo = r * H

    pltpu.sync_copy(wbuf.at[pl.ds(0, APT)], accum.at[pl.ds(s * APT, APT)])
    if deg_part is not None:
        ones, dbuf, dacc = deg_part

        @pl.when(s == 0)
        def _():
            pltpu.sync_copy(dbuf, dacc)

    plsc.subcore_barrier()

    base = s * TPB

    def _drain(k):
        pltpu.make_async_copy(rows[k], accum.at[didx[k]], ssem[k]).wait()
        if deg_part is not None:
            ones, _, dacc = deg_part
            pltpu.make_async_copy(ones, dacc.at[didx[k]], ssem[k]).wait()

    def _load_idx(b, k):
        pltpu.async_copy(src_hbm.at[b], sidx[k], esem[k])
        pltpu.async_copy(dst_hbm.at[b], didxr[k], esem[k])

    def _wait_idx(b, k):
        pltpu.make_async_copy(src_hbm.at[b], sidx[k], esem[k]).wait()
        pltpu.make_async_copy(dst_hbm.at[b], didxr[k], esem[k]).wait()

    for k in range(NBUF):
        _load_idx(base + k, k)

    def batch(i, carry):
        for k in range(NBUF):
            b = i * NBUF + k

            @pl.when(i > 0)
            def _():
                _drain(k)

            _wait_idx(base + b, k)
            for j in range(B // 16):
                sl = pl.ds(j * 16, 16)
                u = didxr[k][sl]
                m = (u >= lo) & (u < lo + H)
                didx[k][sl] = jnp.where(m, u - lo, DLOC)
            pltpu.async_copy(x_hbm.at[sidx[k]], rows[k], gsem[k])

        for k in range(NBUF):
            b = i * NBUF + k
            pltpu.make_async_copy(x_hbm.at[sidx[k]], rows[k],
                                  gsem[k]).wait()

            @pl.when(b + NBUF < TPB)
            def _():
                _load_idx(base + b + NBUF, k)

            desc = pltpu.make_async_copy(rows[k], accum.at[didx[k]], ssem[k])
            desc.start(add=True)
            if deg_part is not None:
                ones, _, dacc = deg_part
                d2 = pltpu.make_async_copy(ones, dacc.at[didx[k]], ssem[k])
                d2.start(add=True)
        return carry

    lax.fori_loop(0, TPB // NBUF, batch, 0)
    for k in range(NBUF):
        _drain(k)

    plsc.subcore_barrier()

    pltpu.sync_copy(accum.at[pl.ds(s * HPT, HPT)], wbuf.at[pl.ds(0, HPT)])
    pltpu.sync_copy(wbuf.at[pl.ds(0, HPT)],
                    out_hbm.at[pl.ds(lo + s * HPT, HPT)])
    if deg_part is not None:
        ones, dbuf, dacc = deg_part
        pltpu.sync_copy(dacc.at[pl.ds(s * HPT, HPT)], dbuf.at[pl.ds(0, HPT)])
        pltpu.sync_copy(dbuf.at[pl.ds(0, HPT)],
                        deg_hbm.at[pl.ds(lo + s * HPT, HPT)])
    plsc.subcore_barrier()
    _zero_2d(wbuf, APT, D)
    if deg_part is not None:
        ones, dbuf, dacc = deg_part
        _zero_1d(dbuf, ACC)


_AGG_SCRATCH = (
    [pltpu.VMEM((B,), jnp.int32) for _ in range(3 * NBUF)]
    + [pltpu.VMEM((B, D), jnp.float32) for _ in range(NBUF)]
    + [pltpu.SemaphoreType.DMA for _ in range(3 * NBUF)]
    + [
        pltpu.VMEM((APT, D), jnp.float32),
        pltpu.VMEM_SHARED((ACC, D), jnp.float32),
    ]
)


@functools.partial(
    pl.kernel,
    out_type=[
        jax.ShapeDtypeStruct((NOUT, D), jnp.float32),
        jax.ShapeDtypeStruct((NOUT,), jnp.float32),
    ],
    mesh=_mesh1,
    scratch_types=_AGG_SCRATCH + [
        pltpu.VMEM((B,), jnp.float32),
        pltpu.VMEM((ACC,), jnp.float32),
        pltpu.VMEM_SHARED((ACC,), jnp.float32),
    ],
)
def _agg_deg(x_hbm, src_hbm, dst_hbm, out_hbm, deg_hbm, *refs):
    sidx = refs[0:NBUF]
    didxr = refs[NBUF:2 * NBUF]
    didx = refs[2 * NBUF:3 * NBUF]
    rows = refs[3 * NBUF:4 * NBUF]
    esem = refs[4 * NBUF:5 * NBUF]
    gsem = refs[5 * NBUF:6 * NBUF]
    ssem = refs[6 * NBUF:7 * NBUF]
    wbuf, accum, ones, dbuf, dacc = refs[7 * NBUF:]

    one = jnp.full((16,), 1.0, jnp.float32)
    for j in range(B // 16):
        ones[pl.ds(j * 16, 16)] = one
    _zero_2d(wbuf, APT, D)
    _zero_1d(dbuf, ACC)
    for r in range(NR):
        _agg_round(r, x_hbm, src_hbm, dst_hbm, out_hbm, deg_hbm, sidx, didxr,
                   didx, rows, esem, gsem, ssem, wbuf, accum,
                   (ones, dbuf, dacc))


@functools.partial(
    pl.kernel,
    out_type=jax.ShapeDtypeStruct((NOUT, D), jnp.float32),
    mesh=_mesh1,
    scratch_types=_AGG_SCRATCH,
)
def _agg(x_hbm, src_hbm, dst_hbm, out_hbm, *refs):
    sidx = refs[0:NBUF]
    didxr = refs[NBUF:2 * NBUF]
    didx = refs[2 * NBUF:3 * NBUF]
    rows = refs[3 * NBUF:4 * NBUF]
    esem = refs[4 * NBUF:5 * NBUF]
    gsem = refs[5 * NBUF:6 * NBUF]
    ssem = refs[6 * NBUF:7 * NBUF]
    wbuf, accum = refs[7 * NBUF:]

    _zero_2d(wbuf, APT, D)
    for r in range(NR):
        _agg_round(r, x_hbm, src_hbm, dst_hbm, out_hbm, None, sidx, didxr,
                   didx, rows, esem, gsem, ssem, wbuf, accum, None)


@functools.partial(
    pl.kernel,
    out_type=jax.ShapeDtypeStruct((EPAD,), jnp.float32),
    mesh=_mesh2,
    scratch_types=(
        [pltpu.VMEM((B,), jnp.int32) for _ in range(2 * NBUF)]
        + [pltpu.VMEM((B,), jnp.float32) for _ in range(3 * NBUF)]
        + [pltpu.SemaphoreType.DMA for _ in range(3 * NBUF)]
    ),
)
def _decode(s_hbm, t_hbm, src_hbm, dst_hbm, out_hbm, *refs):
    sidx = refs[0:NBUF]
    didx = refs[NBUF:2 * NBUF]
    sg = refs[2 * NBUF:3 * NBUF]
    tg = refs[3 * NBUF:4 * NBUF]
    ob = refs[4 * NBUF:5 * NBUF]
    esem = refs[5 * NBUF:6 * NBUF]
    gsem = refs[6 * NBUF:7 * NBUF]
    osem = refs[7 * NBUF:8 * NBUF]

    c = lax.axis_index("c")
    s = lax.axis_index("s")
    tpb = NBATCH // (NC * NS)
    base = (c * NS + s) * tpb

    def _load_idx(b, k):
        pltpu.async_copy(src_hbm.at[b], sidx[k], esem[k])
        pltpu.async_copy(dst_hbm.at[b], didx[k], esem[k])

    def _wait_idx(b, k):
        pltpu.make_async_copy(src_hbm.at[b], sidx[k], esem[k]).wait()
        pltpu.make_async_copy(dst_hbm.at[b], didx[k], esem[k]).wait()

    for k in range(NBUF):
        _load_idx(base + k, k)

    def batch(i, carry):
        for k in range(NBUF):
            b = i * NBUF + k
            _wait_idx(base + b, k)
            pltpu.async_copy(s_hbm.at[sidx[k]], sg[k], gsem[k])
            pltpu.async_copy(t_hbm.at[didx[k]], tg[k], gsem[k])
        for k in range(NBUF):
            b = i * NBUF + k
            pltpu.make_async_copy(s_hbm.at[sidx[k]], sg[k], gsem[k]).wait()
            pltpu.make_async_copy(t_hbm.at[didx[k]], tg[k], gsem[k]).wait()

            @pl.when(i > 0)
            def _():
                pltpu.make_async_copy(
                    ob[k], out_hbm.at[pl.ds(0, B)], osem[k]).wait()

            for j in range(B // 16):
                sl = pl.ds(j * 16, 16)
                v = sg[k][sl] + tg[k][sl]
                ob[k][sl] = 1.0 / (1.0 + jnp.exp(-v))

            @pl.when(b + NBUF < tpb)
            def _():
                _load_idx(base + b + NBUF, k)
            pltpu.async_copy(ob[k], out_hbm.at[pl.ds((base + b) * B, B)],
                             osem[k])
        return carry

    lax.fori_loop(0, tpb // NBUF, batch, 0)
    for k in range(NBUF):
        pltpu.make_async_copy(ob[k], out_hbm.at[pl.ds(0, B)], osem[k]).wait()


RB = 2000
GRID = N // RB


def _tc1_body(p_ref, dg_ref, x_ref, wl_ref, wr_ref, bl_ref, o_ref):
    inv = 1.0 / jnp.maximum(dg_ref[...], 1.0)
    agg = p_ref[...] * inv
    h = lax.dot_general(agg, wl_ref[...], (((1,), (1,)), ((), ())),
                        preferred_element_type=jnp.float32)
    h = h + bl_ref[...]
    h = h + lax.dot_general(x_ref[...], wr_ref[...], (((1,), (1,)), ((), ())),
                            preferred_element_type=jnp.float32)
    o_ref[...] = jnp.maximum(h, 0.0)


def _tc1(p, dg, x, Wl1, bl1, Wr1):
    return pl.pallas_call(
        _tc1_body,
        grid=(GRID,),
        in_specs=[
            pl.BlockSpec((RB, D), lambda i: (i, 0)),
            pl.BlockSpec((RB, 1), lambda i: (i, 0)),
            pl.BlockSpec((RB, D), lambda i: (i, 0)),
            pl.BlockSpec((D, D), lambda i: (0, 0)),
            pl.BlockSpec((D, D), lambda i: (0, 0)),
            pl.BlockSpec((1, D), lambda i: (0, 0)),
        ],
        out_specs=pl.BlockSpec((RB, D), lambda i: (i, 0)),
        out_shape=jax.ShapeDtypeStruct((N, D), jnp.float32),
    )(p, dg, x, Wl1, Wr1, bl1)


def _tc2_body(p_ref, dg_ref, h_ref, wl_ref, wr_ref, bl_ref, wd_ref, bv_ref,
              o_ref):
    inv = 1.0 / jnp.maximum(dg_ref[...], 1.0)
    agg = p_ref[...] * inv
    z = lax.dot_general(agg, wl_ref[...], (((1,), (1,)), ((), ())),
                        preferred_element_type=jnp.float32)
    z = z + bl_ref[...]
    z = z + lax.dot_general(h_ref[...], wr_ref[...], (((1,), (1,)), ((), ())),
                            preferred_element_type=jnp.float32)
    o_ref[...] = lax.dot_general(z, wd_ref[...], (((1,), (0,)), ((), ())),
                                 preferred_element_type=jnp.float32) + bv_ref[...]


def _tc2(p, dg, h, Wl2, bl2, Wr2, wd, bv):
    return pl.pallas_call(
        _tc2_body,
        grid=(GRID,),
        in_specs=[
            pl.BlockSpec((RB, D), lambda i: (i, 0)),
            pl.BlockSpec((RB, 1), lambda i: (i, 0)),
            pl.BlockSpec((RB, D), lambda i: (i, 0)),
            pl.BlockSpec((D, D), lambda i: (0, 0)),
            pl.BlockSpec((D, D), lambda i: (0, 0)),
            pl.BlockSpec((1, D), lambda i: (0, 0)),
            pl.BlockSpec((D, 2), lambda i: (0, 0)),
            pl.BlockSpec((1, 2), lambda i: (0, 0)),
        ],
        out_specs=pl.BlockSpec((RB, 2), lambda i: (i, 0)),
        out_shape=jax.ShapeDtypeStruct((N, 2), jnp.float32),
    )(p, dg, h, Wl2, Wr2, bl2, wd, bv)


def kernel(x, edge_index, Wl1, bl1, Wr1, Wl2, bl2, Wr2, Wlin, blin):
    src = edge_index[0].astype(jnp.int32)
    dst = edge_index[1].astype(jnp.int32)
    pad = EPAD - E
    src2d = jnp.concatenate([src, jnp.zeros((pad,), jnp.int32)]).reshape(
        NBATCH, B)
    dst2d = jnp.concatenate([dst, jnp.full((pad,), DUMMY, jnp.int32)]).reshape(
        NBATCH, B)

    p1, deg = _agg_deg(x, src2d, dst2d)
    dg = deg[:N].reshape(N, 1)
    h = _tc1(p1, dg, x, Wl1, bl1.reshape(1, D), Wr1)

    p2 = _agg(h, src2d, dst2d)
    wd = jnp.stack([Wlin[0, :D], Wlin[0, D:]], axis=1)
    bv = jnp.stack([blin[0], jnp.zeros((), jnp.float32)]).reshape(1, 2)
    st = _tc2(p2, dg, h, Wl2, bl2.reshape(1, D), Wr2, wd, bv)

    s_tab = jnp.pad(st[:, 0], (0, NOUT + B - N))
    t_tab = jnp.pad(st[:, 1], (0, NOUT + B - N))
    logits = _decode(s_tab, t_tab, src2d, dst2d)
    return logits[:E]
